# baseline (speedup 1.0000x reference)
import jax
import jax.numpy as jnp
from jax.experimental import pallas as pl
from jax.experimental.pallas import tpu as pltpu

_N = 320000
_K = 64
_D = 128
_BLOCK = 16000

_BALANCE_W = 0.1
_SEPARATION_W = 0.1
_COMPACTNESS_W = 0.1


def _dot(x, y, dims):
    return jax.lax.dot_general(x, y, (dims, ((), ())),
                               preferred_element_type=jnp.float32)


def _phase1_body(a_ref, hard_ref, probs_ref, acc_ref):
    step = pl.program_id(0)
    nsteps = pl.num_programs(0)

    @pl.when(step == 0)
    def _init():
        acc_ref[...] = jnp.zeros_like(acc_ref)

    a = a_ref[...]
    acc_ref[...] += jnp.sum(a, axis=0, keepdims=True)

    m = jnp.max(a, axis=1, keepdims=True)
    colf = jax.lax.broadcasted_iota(jnp.int32, a.shape, 1).astype(jnp.float32)
    hard_ref[...] = jnp.min(jnp.where(a == m, colf, float(_K)),
                            axis=1, keepdims=True)

    @pl.when(step == nsteps - 1)
    def _final():
        probs_ref[...] = acc_ref[...]


def _phase2_body(z_ref, hard_ref, c_ref, psum_ref, out_ref, seg_ref, cnt_ref):
    step = pl.program_id(0)
    nsteps = pl.num_programs(0)

    @pl.when(step == 0)
    def _init():
        seg_ref[...] = jnp.zeros_like(seg_ref)
        cnt_ref[...] = jnp.zeros_like(cnt_ref)

    z = z_ref[...]
    c = c_ref[...]
    hard = hard_ref[...]

    colf = jax.lax.broadcasted_iota(
        jnp.int32, (z.shape[0], _K), 1).astype(jnp.float32)
    onehot = jnp.where(colf == hard, 1.0, 0.0)

    ones_d = jnp.ones((_D, 1), jnp.float32)
    ones_dk = jnp.ones((_D, _K), jnp.float32)
    ones_1d = jnp.ones((1, _D), jnp.float32)

    zc = _dot(z, c, ((1,), (1,)))
    zsqk = _dot(z * z, ones_dk, ((1,), (0,)))
    csq_row = _dot(ones_1d, c * c, ((1,), (1,)))
    w = onehot * (zsqk + (csq_row - 2.0 * zc))
    wc = jnp.maximum(w, 0.0)
    pdm = wc * jax.lax.rsqrt(wc + 1e-12)

    seg_ref[...] += jnp.sum(pdm, axis=0, keepdims=True)
    cnt_ref[...] += jnp.sum(onehot, axis=0, keepdims=True)

    @pl.when(step == nsteps - 1)
    def _final():
        probs = psum_ref[0, :] / _N
        seg = seg_ref[0, :]
        cnt = cnt_ref[0, :]

        t = 1.0 / _K
        balance = jnp.sum(t * (jnp.log(t) - jnp.log(probs + 1e-8)))

        cc = _dot(c, c, ((1,), (1,)))
        csq_col = _dot(c * c, ones_d, ((1,), (0,)))
        d2 = jnp.maximum(csq_col + csq_row - 2.0 * cc, 0.0)
        dist = jnp.sqrt(d2)
        r = jax.lax.broadcasted_iota(jnp.int32, (_K, _K), 0)
        q = jax.lax.broadcasted_iota(jnp.int32, (_K, _K), 1)
        separation = -jnp.sum(jnp.where(r != q, dist, 0.0)) / (_K * (_K - 1))

        nonempty = cnt > 0
        means = seg / jnp.where(nonempty, cnt, 1.0)
        nn = jnp.sum(nonempty.astype(jnp.float32))
        compact = jnp.where(
            nn > 0,
            jnp.sum(jnp.where(nonempty, means, 0.0)) / jnp.maximum(nn, 1.0),
            0.0)

        aux = _BALANCE_W * balance + _SEPARATION_W * separation \
            + _COMPACTNESS_W * compact
        mean_p = jnp.mean(probs)
        cbal = jnp.sqrt(jnp.sum((probs - mean_p) ** 2) / (_K - 1))

        lane = jax.lax.broadcasted_iota(jnp.int32, (1, 8), 1)
        vec = jnp.zeros((1, 8), jnp.float32)
        vec = jnp.where(lane == 0, aux, vec)
        vec = jnp.where(lane == 1, balance, vec)
        vec = jnp.where(lane == 2, separation, vec)
        vec = jnp.where(lane == 3, compact, vec)
        vec = jnp.where(lane == 4, cbal, vec)
        out_ref[...] = vec


def kernel(latent_z, cluster_assignments, cluster_centers):
    hard, psum = pl.pallas_call(
        _phase1_body,
        grid=(_N // _BLOCK,),
        in_specs=[
            pl.BlockSpec((_BLOCK, _K), lambda i: (i, 0)),
        ],
        out_specs=[
            pl.BlockSpec((_BLOCK, 1), lambda i: (i, 0)),
            pl.BlockSpec((1, _K), lambda i: (0, 0)),
        ],
        out_shape=[
            jax.ShapeDtypeStruct((_N, 1), jnp.float32),
            jax.ShapeDtypeStruct((1, _K), jnp.float32),
        ],
        scratch_shapes=[
            pltpu.VMEM((1, _K), jnp.float32),
        ],
        compiler_params=pltpu.CompilerParams(
            dimension_semantics=("arbitrary",)),
    )(cluster_assignments)

    out = pl.pallas_call(
        _phase2_body,
        grid=(_N // _BLOCK,),
        in_specs=[
            pl.BlockSpec((_BLOCK, _D), lambda i: (i, 0)),
            pl.BlockSpec((_BLOCK, 1), lambda i: (i, 0)),
            pl.BlockSpec((_K, _D), lambda i: (0, 0)),
            pl.BlockSpec((1, _K), lambda i: (0, 0)),
        ],
        out_specs=pl.BlockSpec((1, 8), lambda i: (0, 0)),
        out_shape=jax.ShapeDtypeStruct((1, 8), jnp.float32),
        scratch_shapes=[
            pltpu.VMEM((1, _K), jnp.float32),
            pltpu.VMEM((1, _K), jnp.float32),
        ],
        compiler_params=pltpu.CompilerParams(
            dimension_semantics=("arbitrary",)),
    )(latent_z, hard, cluster_centers, psum)
    o = out[0]
    return (o[0], o[1], o[2], o[3], o[4])


# speedup vs baseline: 1.3267x; 1.3267x over previous
import jax
import jax.numpy as jnp
from jax.experimental import pallas as pl
from jax.experimental.pallas import tpu as pltpu

_N = 320000
_K = 64
_D = 128
_BLOCK = 16000

_BALANCE_W = 0.1
_SEPARATION_W = 0.1
_COMPACTNESS_W = 0.1


def _dot(x, y, dims):
    return jax.lax.dot_general(x, y, (dims, ((), ())),
                               preferred_element_type=jnp.float32)


def _body(z_ref, a_ref, c_ref, out_ref, probs_ref, seg_ref, cnt_ref):
    step = pl.program_id(0)
    nsteps = pl.num_programs(0)

    @pl.when(step == 0)
    def _init():
        probs_ref[...] = jnp.zeros_like(probs_ref)
        seg_ref[...] = jnp.zeros_like(seg_ref)
        cnt_ref[...] = jnp.zeros_like(cnt_ref)

    a = a_ref[...]
    z = z_ref[...]
    c = c_ref[...]

    probs_ref[...] += jnp.sum(a, axis=0, keepdims=True)

    m = jnp.max(a, axis=1, keepdims=True)
    colf = jax.lax.broadcasted_iota(jnp.int32, a.shape, 1).astype(jnp.float32)
    hardf = jnp.min(jnp.where(a == m, colf, float(_K)),
                    axis=1, keepdims=True)
    onehot = jnp.where(colf == hardf, 1.0, 0.0)

    ones_d = jnp.ones((_D, 1), jnp.float32)
    ones_dk = jnp.ones((_D, _K), jnp.float32)
    ones_1d = jnp.ones((1, _D), jnp.float32)

    zc = _dot(z, c, ((1,), (1,)))
    zsqk = _dot(z * z, ones_dk, ((1,), (0,)))
    csq_row = _dot(ones_1d, c * c, ((1,), (1,)))
    w = onehot * (zsqk + (csq_row - 2.0 * zc))
    wc = jnp.maximum(w, 0.0)
    pdm = wc * jax.lax.rsqrt(wc + 1e-12)

    seg_ref[...] += jnp.sum(pdm, axis=0, keepdims=True)
    cnt_ref[...] += jnp.sum(onehot, axis=0, keepdims=True)

    @pl.when(step == nsteps - 1)
    def _final():
        probs = probs_ref[0, :] / _N
        seg = seg_ref[0, :]
        cnt = cnt_ref[0, :]

        t = 1.0 / _K
        balance = jnp.sum(t * (jnp.log(t) - jnp.log(probs + 1e-8)))

        cc = _dot(c, c, ((1,), (1,)))
        csq_col = _dot(c * c, ones_d, ((1,), (0,)))
        d2 = csq_col + csq_row - 2.0 * cc
        d2 = jnp.maximum(d2, 0.0)
        dist = jnp.sqrt(d2)
        r = jax.lax.broadcasted_iota(jnp.int32, (_K, _K), 0)
        q = jax.lax.broadcasted_iota(jnp.int32, (_K, _K), 1)
        separation = -jnp.sum(jnp.where(r != q, dist, 0.0)) / (_K * (_K - 1))

        nonempty = cnt > 0
        means = seg / jnp.where(nonempty, cnt, 1.0)
        nn = jnp.sum(nonempty.astype(jnp.float32))
        compact = jnp.where(
            nn > 0,
            jnp.sum(jnp.where(nonempty, means, 0.0)) / jnp.maximum(nn, 1.0),
            0.0)

        aux = _BALANCE_W * balance + _SEPARATION_W * separation \
            + _COMPACTNESS_W * compact
        mean_p = jnp.mean(probs)
        cbal = jnp.sqrt(jnp.sum((probs - mean_p) ** 2) / (_K - 1))

        lane = jax.lax.broadcasted_iota(jnp.int32, (1, 8), 1)
        vec = jnp.zeros((1, 8), jnp.float32)
        vec = jnp.where(lane == 0, aux, vec)
        vec = jnp.where(lane == 1, balance, vec)
        vec = jnp.where(lane == 2, separation, vec)
        vec = jnp.where(lane == 3, compact, vec)
        vec = jnp.where(lane == 4, cbal, vec)
        out_ref[...] = vec


def kernel(latent_z, cluster_assignments, cluster_centers):
    out = pl.pallas_call(
        _body,
        grid=(_N // _BLOCK,),
        in_specs=[
            pl.BlockSpec((_BLOCK, _D), lambda i: (i, 0)),
            pl.BlockSpec((_BLOCK, _K), lambda i: (i, 0)),
            pl.BlockSpec((_K, _D), lambda i: (0, 0)),
        ],
        out_specs=pl.BlockSpec((1, 8), lambda i: (0, 0)),
        out_shape=jax.ShapeDtypeStruct((1, 8), jnp.float32),
        scratch_shapes=[
            pltpu.VMEM((1, _K), jnp.float32),
            pltpu.VMEM((1, _K), jnp.float32),
            pltpu.VMEM((1, _K), jnp.float32),
        ],
        compiler_params=pltpu.CompilerParams(
            dimension_semantics=("arbitrary",)),
    )(latent_z, cluster_assignments, cluster_centers)
    o = out[0]
    return (o[0], o[1], o[2], o[3], o[4])
